# baseline (speedup 1.0000x reference)
import functools

import jax
import jax.numpy as jnp
from jax import lax
from jax.experimental import pallas as pl
from jax.experimental.pallas import tpu as pltpu
from jax.experimental.pallas import tpu_sc as plsc

_B = 16384
_D = 32
_NTC = 7813
_NC = 2
_NS = 16
_NW = _NC * _NS
_BPW = _B // _NW
_NQ = 245
_NWAVE = 16
_LCAP = 768
_BCAP = 128
_SROWS = _B + 8


def _gather_body(uidx_hbm, iidx_hbm, ut_hbm, it_hbm, stagu_hbm, stagi_hbm,
                 idx_v, lst_r, lst_id, rbuf, idbuf, ring, blk,
                 bcnt_s, semf, sems):
    wid = lax.axis_index("s") * _NC + lax.axis_index("c")
    lanes = lax.iota(jnp.int32, 16)

    for side in range(2):
        idx_hbm = (uidx_hbm, iidx_hbm)[side]
        tab_hbm = (ut_hbm, it_hbm)[side]
        stag_hbm = (stagu_hbm, stagi_hbm)[side]

        pltpu.sync_copy(idx_hbm, idx_v)

        def a1(g, cnt):
            vec = idx_v[pl.ds(g * 16, 16)]
            tc = vec >> 7
            m = (tc & 31) == wid
            plsc.store_compressed(lst_r.at[pl.ds(cnt, 16)], vec, mask=m)
            plsc.store_compressed(lst_id.at[pl.ds(cnt, 16)],
                                  g * 16 + lanes, mask=m)
            return cnt + jnp.sum(m.astype(jnp.int32))

        cnt = lax.fori_loop(0, _B // 16, a1, jnp.int32(0))

        for bb in range(_NWAVE):
            bcnt_s[bb] = 0

        def a2(v, carry):
            rvec = lst_r[pl.ds(v * 16, 16)]
            idvec = lst_id[pl.ds(v * 16, 16)]
            valid = (v * 16 + lanes) < cnt
            for bb in range(_NWAVE):
                m = valid & ((rvec >> 16) == bb)
                c0 = bcnt_s[bb]
                plsc.store_compressed(rbuf.at[bb, pl.ds(c0, 16)], rvec, mask=m)
                plsc.store_compressed(idbuf.at[bb, pl.ds(c0, 16)], idvec, mask=m)
                bcnt_s[bb] = c0 + jnp.sum(m.astype(jnp.int32))
            return carry

        lax.fori_loop(0, (cnt + 15) >> 4, a2, jnp.int32(0))

        def wave(t, carry):
            for qq in range(_NWAVE):
                tc_s = ((t * 16 + qq) << 5) | wid

                @pl.when(tc_s < _NTC)
                def _():
                    off = pl.multiple_of(tc_s * 128, 128)
                    pltpu.async_copy(tab_hbm.at[:, pl.ds(off, 128)],
                                     ring.at[qq], semf)

            for qq in range(_NWAVE):
                tc_s = ((t * 16 + qq) << 5) | wid

                @pl.when(tc_s < _NTC)
                def _():
                    pltpu.make_async_copy(tab_hbm.at[:, pl.ds(0, 128)],
                                          ring.at[qq], semf).wait()

            bn = bcnt_s[t]
            for v in range(_BCAP // 16):

                @pl.when(v * 16 < bn)
                def _():
                    rvec = rbuf[t, pl.ds(v * 16, 16)]
                    idvec = idbuf[t, pl.ds(v * 16, 16)]
                    valid = (v * 16 + lanes) < bn
                    qloc = (rvec >> 12) & 15
                    colv = rvec & 127
                    for d in range(_D):
                        dv = jnp.full((16,), d, jnp.int32)
                        val = plsc.load_gather(ring, [qloc, dv, colv])
                        plsc.store_scatter(blk.at[v], [lanes, dv], val)
                    ids = jnp.where(valid, idvec, _B)
                    pltpu.async_copy(blk.at[v], stag_hbm.at[ids], sems)

            def sdrain(v2, c2):
                pltpu.make_async_copy(stag_hbm.at[pl.ds(0, 16)],
                                      blk.at[0], sems).wait()
                return c2

            lax.fori_loop(0, (bn + 15) >> 4, sdrain, jnp.int32(0))
            return carry

        lax.fori_loop(0, _NWAVE, wave, jnp.int32(0))


def _dot_body(stagu_hbm, stagi_hbm, out_hbm, ublk, iblk, out_v, semd):
    wid = lax.axis_index("s") * _NC + lax.axis_index("c")
    base = wid * _BPW
    lanes = lax.iota(jnp.int32, 16)

    def chunk(c, carry):
        row0 = base + c * 64
        cu = pltpu.async_copy(stagu_hbm.at[pl.ds(row0, 64)], ublk, semd)
        ci = pltpu.async_copy(stagi_hbm.at[pl.ds(row0, 64)], iblk, semd)
        cu.wait()
        ci.wait()
        for g in range(4):
            rowv = g * 16 + lanes
            acc = jnp.zeros((16,), jnp.float32)
            for d in range(_D):
                colv = (lanes + d) & (_D - 1)
                acc = acc + (plsc.load_gather(ublk, [rowv, colv]) *
                             plsc.load_gather(iblk, [rowv, colv]))
            out_v[pl.ds(c * 64 + g * 16, 16)] = acc
        return carry

    lax.fori_loop(0, _BPW // 64, chunk, jnp.int32(0))
    pltpu.sync_copy(out_v, out_hbm.at[wid])


@jax.jit
def _sc_call(uidx, iidx, ut, it):
    mesh = plsc.VectorSubcoreMesh(core_axis_name="c", subcore_axis_name="s")
    gather_fn = functools.partial(
        pl.kernel,
        mesh=mesh,
        out_type=(
            jax.ShapeDtypeStruct((_SROWS, 128), jnp.float32),
            jax.ShapeDtypeStruct((_SROWS, 128), jnp.float32),
        ),
        scratch_types=[
            pltpu.VMEM((_B,), jnp.int32),
            pltpu.VMEM((_LCAP,), jnp.int32),
            pltpu.VMEM((_LCAP,), jnp.int32),
            pltpu.VMEM((_NWAVE, _BCAP), jnp.int32),
            pltpu.VMEM((_NWAVE, _BCAP), jnp.int32),
            pltpu.VMEM((_NWAVE, _D, 128), jnp.float32),
            pltpu.VMEM((_BCAP // 16, 16, 128), jnp.float32),
            pltpu.SMEM((_NWAVE,), jnp.int32),
            pltpu.SemaphoreType.DMA,
            pltpu.SemaphoreType.DMA,
        ],
        compiler_params=pltpu.CompilerParams(needs_layout_passes=False),
    )(_gather_body)
    stag_u, stag_i = gather_fn(uidx, iidx, ut, it)

    dot_fn = functools.partial(
        pl.kernel,
        mesh=mesh,
        out_type=jax.ShapeDtypeStruct((_NW, _BPW), jnp.float32),
        scratch_types=[
            pltpu.VMEM((64, 128), jnp.float32),
            pltpu.VMEM((64, 128), jnp.float32),
            pltpu.VMEM((_BPW,), jnp.float32),
            pltpu.SemaphoreType.DMA,
        ],
        compiler_params=pltpu.CompilerParams(needs_layout_passes=False),
    )(_dot_body)
    return dot_fn(stag_u, stag_i)


def kernel(input_tensor, user_table, item_table):
    uidx = input_tensor[:, 0].astype(jnp.int32)
    iidx = input_tensor[:, 1].astype(jnp.int32)
    out = _sc_call(uidx, iidx, user_table.T, item_table.T)
    return out.reshape(_B, 1)


# speedup vs baseline: 1.3073x; 1.3073x over previous
import functools

import jax
import jax.numpy as jnp
from jax import lax
from jax.experimental import pallas as pl
from jax.experimental.pallas import tpu as pltpu
from jax.experimental.pallas import tpu_sc as plsc

_B = 16384
_D = 32
_NTC = 7813
_NC = 2
_NS = 16
_NW = _NC * _NS
_BPW = _B // _NW
_NWAVES = 31
_NBKT = 32
_LCAP = 1344
_BCAP = 128
_SROWS = _B + 8


def _gather_body(uidx_hbm, iidx_hbm, ut_hbm, it_hbm, stagu_hbm, stagi_hbm,
                 idx_v, lst_r, lst_id, rbuf, idbuf, ring, blk,
                 bcnt_s, semf, sems):
    side = lax.axis_index("c")
    sid = lax.axis_index("s")
    lanes = lax.iota(jnp.int32, 16)

    def flow(idx_hbm, tab_hbm, stag_hbm):
        pltpu.sync_copy(idx_hbm, idx_v)

        def a1(g, cnt):
            vec = idx_v[pl.ds(g * 16, 16)]
            tc = vec >> 7
            m = (tc & 15) == sid
            plsc.store_compressed(lst_r.at[pl.ds(cnt, 16)], vec, mask=m)
            plsc.store_compressed(lst_id.at[pl.ds(cnt, 16)],
                                  g * 16 + lanes, mask=m)
            return cnt + jnp.sum(m.astype(jnp.int32))

        cnt = lax.fori_loop(0, _B // 16, a1, jnp.int32(0))

        for bb in range(_NBKT):
            bcnt_s[bb] = 0

        def a2(v, carry):
            rvec = lst_r[pl.ds(v * 16, 16)]
            idvec = lst_id[pl.ds(v * 16, 16)]
            valid = (v * 16 + lanes) < cnt
            for bb in range(_NBKT - 1):
                m = valid & ((rvec >> 15) == bb)
                c0 = bcnt_s[bb]
                plsc.store_compressed(rbuf.at[bb, pl.ds(c0, 16)], rvec, mask=m)
                plsc.store_compressed(idbuf.at[bb, pl.ds(c0, 16)], idvec,
                                      mask=m)
                bcnt_s[bb] = c0 + jnp.sum(m.astype(jnp.int32))
            return carry

        lax.fori_loop(0, (cnt + 15) >> 4, a2, jnp.int32(0))

        def wave(t, carry):
            for qq in range(16):
                tc_s = ((t * 16 + qq) << 4) | sid

                @pl.when(tc_s < _NTC)
                def _():
                    off = pl.multiple_of(tc_s * 128, 128)
                    pltpu.async_copy(tab_hbm.at[:, pl.ds(off, 128)],
                                     ring.at[qq], semf)

            for qq in range(16):
                tc_s = ((t * 16 + qq) << 4) | sid

                @pl.when(tc_s < _NTC)
                def _():
                    pltpu.make_async_copy(tab_hbm.at[:, pl.ds(0, 128)],
                                          ring.at[qq], semf).wait()

            bn = bcnt_s[t]
            for v in range(_BCAP // 16):

                @pl.when(v * 16 < bn)
                def _():
                    rvec = rbuf[t, pl.ds(v * 16, 16)]
                    idvec = idbuf[t, pl.ds(v * 16, 16)]
                    valid = (v * 16 + lanes) < bn
                    qloc = (rvec >> 11) & 15
                    colv = rvec & 127
                    rot = idvec & 15
                    for d in range(_D):
                        dv = jnp.full((16,), d, jnp.int32)
                        val = plsc.load_gather(ring, [qloc, dv, colv])
                        plsc.store_scatter(blk.at[v], [lanes, (dv + rot) & 31],
                                           val)
                    ids = jnp.where(valid, idvec, _B)
                    pltpu.async_copy(blk.at[v], stag_hbm.at[ids], sems)

            def sdrain(v2, c2):
                pltpu.make_async_copy(stag_hbm.at[pl.ds(0, 16)],
                                      blk.at[0], sems).wait()
                return c2

            lax.fori_loop(0, (bn + 15) >> 4, sdrain, jnp.int32(0))
            return carry

        lax.fori_loop(0, _NWAVES, wave, jnp.int32(0))

    @pl.when(side == 0)
    def _():
        flow(uidx_hbm, ut_hbm, stagu_hbm)

    @pl.when(side == 1)
    def _():
        flow(iidx_hbm, it_hbm, stagi_hbm)


def _dot_body(stagu_hbm, stagi_hbm, out_hbm, ublk, iblk, out_v, semd):
    wid = lax.axis_index("s") * _NC + lax.axis_index("c")
    base = wid * _BPW
    lanes = lax.iota(jnp.int32, 16)

    def chunk(c, carry):
        row0 = base + c * 64
        cu = pltpu.async_copy(stagu_hbm.at[pl.ds(row0, 64)], ublk, semd)
        ci = pltpu.async_copy(stagi_hbm.at[pl.ds(row0, 64)], iblk, semd)
        cu.wait()
        ci.wait()
        for g in range(4):
            rowv = g * 16 + lanes
            acc = jnp.zeros((16,), jnp.float32)
            for d in range(_D):
                colv = (lanes + d) & (_D - 1)
                acc = acc + (plsc.load_gather(ublk, [rowv, colv]) *
                             plsc.load_gather(iblk, [rowv, colv]))
            out_v[pl.ds(c * 64 + g * 16, 16)] = acc
        return carry

    lax.fori_loop(0, _BPW // 64, chunk, jnp.int32(0))
    pltpu.sync_copy(out_v, out_hbm.at[wid])


@jax.jit
def _sc_call(uidx, iidx, ut, it):
    mesh = plsc.VectorSubcoreMesh(core_axis_name="c", subcore_axis_name="s")
    gather_fn = functools.partial(
        pl.kernel,
        mesh=mesh,
        out_type=(
            jax.ShapeDtypeStruct((_SROWS, 128), jnp.float32),
            jax.ShapeDtypeStruct((_SROWS, 128), jnp.float32),
        ),
        scratch_types=[
            pltpu.VMEM((_B,), jnp.int32),
            pltpu.VMEM((_LCAP,), jnp.int32),
            pltpu.VMEM((_LCAP,), jnp.int32),
            pltpu.VMEM((_NBKT, _BCAP), jnp.int32),
            pltpu.VMEM((_NBKT, _BCAP), jnp.int32),
            pltpu.VMEM((16, _D, 128), jnp.float32),
            pltpu.VMEM((_BCAP // 16, 16, 128), jnp.float32),
            pltpu.SMEM((_NBKT,), jnp.int32),
            pltpu.SemaphoreType.DMA,
            pltpu.SemaphoreType.DMA,
        ],
        compiler_params=pltpu.CompilerParams(needs_layout_passes=False),
    )(_gather_body)
    stag_u, stag_i = gather_fn(uidx, iidx, ut, it)

    dot_fn = functools.partial(
        pl.kernel,
        mesh=mesh,
        out_type=jax.ShapeDtypeStruct((_NW, _BPW), jnp.float32),
        scratch_types=[
            pltpu.VMEM((64, 128), jnp.float32),
            pltpu.VMEM((64, 128), jnp.float32),
            pltpu.VMEM((_BPW,), jnp.float32),
            pltpu.SemaphoreType.DMA,
        ],
        compiler_params=pltpu.CompilerParams(needs_layout_passes=False),
    )(_dot_body)
    return dot_fn(stag_u, stag_i)


def kernel(input_tensor, user_table, item_table):
    uidx = input_tensor[:, 0].astype(jnp.int32)
    iidx = input_tensor[:, 1].astype(jnp.int32)
    out = _sc_call(uidx, iidx, user_table.T, item_table.T)
    return out.reshape(_B, 1)


# speedup vs baseline: 2.3791x; 1.8198x over previous
import functools

import jax
import jax.numpy as jnp
from jax import lax
from jax.experimental import pallas as pl
from jax.experimental.pallas import tpu as pltpu
from jax.experimental.pallas import tpu_sc as plsc

_B = 16384
_D = 32
_NC = 2
_NS = 16
_NW = _NC * _NS
_BPW = _B // _NW
_CH = 4
_NCHUNK = _BPW // _CH


def _sc_body(uidx_hbm, iidx_hbm, ut_hbm, it_hbm, out_hbm,
             uidx_v, iidx_v, ubuf, ibuf, out_v,
             sem0, sem1):
    wid = lax.axis_index("s") * _NC + lax.axis_index("c")

    pltpu.sync_copy(uidx_hbm.at[wid], uidx_v)
    pltpu.sync_copy(iidx_hbm.at[wid], iidx_v)

    lanes0 = lax.iota(jnp.int32, 16)

    def scalar_idx(idx_v, item):
        vec = idx_v[pl.ds((item // 16) * 16, 16)]
        return jnp.sum(jnp.where(lanes0 == (item & 15), vec, 0))

    def fire(c, pb, sem):
        for j in range(_CH):
            item = c * _CH + j
            ur = scalar_idx(uidx_v, item)
            cu = pl.multiple_of((ur >> 7) * 128, 128)
            pltpu.async_copy(ut_hbm.at[:, pl.ds(cu, 128)],
                             ubuf.at[pb, j], sem)
            ir = scalar_idx(iidx_v, item)
            ci = pl.multiple_of((ir >> 7) * 128, 128)
            pltpu.async_copy(it_hbm.at[:, pl.ds(ci, 128)],
                             ibuf.at[pb, j], sem)

    def drain(pb, sem):
        for j in range(_CH):
            pltpu.make_async_copy(ut_hbm.at[:, pl.ds(0, 128)],
                                  ubuf.at[pb, j], sem).wait()
            pltpu.make_async_copy(it_hbm.at[:, pl.ds(0, 128)],
                                  ibuf.at[pb, j], sem).wait()

    lanes = lax.iota(jnp.int32, 16)

    def compute(c, pb, acc):
        pbv = jnp.full((16,), pb, jnp.int32)
        for j in range(_CH):
            item = c * _CH + j
            jv = jnp.full((16,), j, jnp.int32)
            cu = jnp.full((16,), scalar_idx(uidx_v, item) & 127, jnp.int32)
            ci = jnp.full((16,), scalar_idx(iidx_v, item) & 127, jnp.int32)
            gu1 = plsc.load_gather(ubuf, [pbv, jv, lanes, cu])
            gu2 = plsc.load_gather(ubuf, [pbv, jv, lanes + 16, cu])
            gi1 = plsc.load_gather(ibuf, [pbv, jv, lanes, ci])
            gi2 = plsc.load_gather(ibuf, [pbv, jv, lanes + 16, ci])
            p = gu1 * gi1 + gu2 * gi2
            s = jnp.sum(p)
            acc = jnp.where(lanes == (item & 15), jnp.full((16,), s), acc)
        return acc

    fire(0, 0, sem0)

    def step(k, acc):
        c0 = 2 * k
        fire(c0 + 1, 1, sem1)
        drain(0, sem0)
        acc = compute(c0, 0, acc)

        @pl.when(c0 + 2 < _NCHUNK)
        def _():
            fire(c0 + 2, 0, sem0)

        drain(1, sem1)
        acc = compute(c0 + 1, 1, acc)

        @pl.when(k % 2 == 1)
        def _():
            out_v[pl.ds((k // 2) * 16, 16)] = acc

        return acc

    lax.fori_loop(0, _NCHUNK // 2, step, jnp.zeros((16,), jnp.float32))
    pltpu.sync_copy(out_v, out_hbm.at[wid])


@jax.jit
def _sc_call(uidx, iidx, ut, it):
    mesh = plsc.VectorSubcoreMesh(core_axis_name="c", subcore_axis_name="s")
    fn = functools.partial(
        pl.kernel,
        mesh=mesh,
        out_type=jax.ShapeDtypeStruct((_NW, _BPW), jnp.float32),
        scratch_types=[
            pltpu.VMEM((_BPW,), jnp.int32),
            pltpu.VMEM((_BPW,), jnp.int32),
            pltpu.VMEM((2, _CH, _D, 128), jnp.float32),
            pltpu.VMEM((2, _CH, _D, 128), jnp.float32),
            pltpu.VMEM((_BPW,), jnp.float32),
            pltpu.SemaphoreType.DMA,
            pltpu.SemaphoreType.DMA,
        ],
        compiler_params=pltpu.CompilerParams(needs_layout_passes=False),
    )(_sc_body)
    return fn(uidx, iidx, ut, it)


def kernel(input_tensor, user_table, item_table):
    uidx = input_tensor[:, 0].astype(jnp.int32).reshape(_NW, _BPW)
    iidx = input_tensor[:, 1].astype(jnp.int32).reshape(_NW, _BPW)
    out = _sc_call(uidx, iidx, user_table.T, item_table.T)
    return out.reshape(_B, 1)
